# Initial kernel scaffold; baseline (speedup 1.0000x reference)
import jax
import jax.numpy as jnp
from jax.experimental import pallas as pl


def kernel(hidden_states, router_weight, gate_up_proj, down_proj, shared_gate_w, shared_up_w, shared_down_w):
    raise NotImplementedError("write your pallas kernel here")


# speedup vs baseline: 1.4867x; 1.4867x over previous
import functools

import jax
import jax.numpy as jnp
from jax import lax
from jax.experimental import pallas as pl
from jax.experimental.pallas import tpu as pltpu
from jax.experimental.pallas import tpu_sc as plsc

HIDDEN = 2048
INTER = 2048
E = 8
T = 2048

BT = 256
NBLK = 16
SROWS = NBLK * BT
CJ = 512
NJ = INTER // CJ

NC, NS = 2, 16
NW = NC * NS
TPW = T // NW
RCH = 16
NCH = TPW // RCH


def _router_body(x_ref, wr_ref, lo_ref, xs_ref, oh_ref):
    xv = x_ref[...]
    lg = jnp.dot(xv, wr_ref[...], preferred_element_type=jnp.float32)
    colv = lax.broadcasted_iota(jnp.int32, lg.shape, 1)
    lgm = jnp.where(colv < E, lg, -3.4e38)
    top = jnp.max(lgm, axis=1, keepdims=True)
    idxcol = jnp.min(jnp.where(lgm == top, colv, 128), axis=1, keepdims=True)
    onehot = colv == idxcol
    score = jax.nn.sigmoid(top)
    lo_ref[...] = lg
    xs_ref[...] = xv * score
    oh_ref[...] = onehot.astype(jnp.int32)


def _run_router(x, wr_pad):
    return pl.pallas_call(
        _router_body,
        grid=(T // BT,),
        in_specs=[
            pl.BlockSpec((BT, HIDDEN), lambda b: (b, 0)),
            pl.BlockSpec((HIDDEN, 128), lambda b: (0, 0)),
        ],
        out_specs=[
            pl.BlockSpec((BT, 128), lambda b: (b, 0)),
            pl.BlockSpec((BT, HIDDEN), lambda b: (b, 0)),
            pl.BlockSpec((BT, 128), lambda b: (b, 0)),
        ],
        out_shape=[
            jax.ShapeDtypeStruct((T, 128), jnp.float32),
            jax.ShapeDtypeStruct((T, HIDDEN), jnp.float32),
            jax.ShapeDtypeStruct((T, 128), jnp.int32),
        ],
    )(x, wr_pad)


def _sched_body(oh_ref, dst_ref, sched_ref):
    oh = oh_ref[...]
    cinc = oh
    k = 1
    while k < T:
        shifted = jnp.concatenate(
            [jnp.zeros((k, 128), jnp.int32), cinc[: T - k, :]], axis=0)
        cinc = cinc + shifted
        k *= 2
    counts = cinc[T - 1 : T, :]
    padded = ((counts + (BT - 1)) // BT) * BT
    erow = lax.broadcasted_iota(jnp.int32, (128, 128), 0)
    ecol = lax.broadcasted_iota(jnp.int32, (128, 128), 1)
    tri = (erow < ecol).astype(jnp.float32)
    pad_off = jnp.dot(padded.astype(jnp.float32), tri,
                      preferred_element_type=jnp.float32).astype(jnp.int32)
    pos = jnp.sum(jnp.where(oh > 0, pad_off + cinc - 1, 0), axis=1,
                  keepdims=True)
    dst_ref[...] = pos

    bnd = (pad_off + padded) // BT
    brow = lax.broadcasted_iota(jnp.int32, (NBLK, 128), 0)
    colv = lax.broadcasted_iota(jnp.int32, (NBLK, 128), 1)
    ge = jnp.logical_and(brow >= bnd, colv < E)
    be = jnp.sum(ge.astype(jnp.int32), axis=1, keepdims=True)
    nu = jnp.sum(padded, axis=1, keepdims=True) // BT
    bvec = lax.broadcasted_iota(jnp.int32, (NBLK, 1), 0)
    be_last = jnp.sum(jnp.where(bvec == nu - 1, be, 0), axis=0,
                      keepdims=True)
    be_fin = jnp.where(bvec < nu, be, be_last)
    sched_ref[...] = jnp.concatenate(
        [jnp.broadcast_to(be_fin, (NBLK, 128)),
         jnp.broadcast_to(nu, (NBLK, 128))], axis=0)


def _run_sched(onehot):
    return pl.pallas_call(
        _sched_body,
        out_shape=[
            jax.ShapeDtypeStruct((T, 1), jnp.int32),
            jax.ShapeDtypeStruct((2 * NBLK, 128), jnp.int32),
        ],
    )(onehot)


def _sc_scatter_body(xsc_hbm, dst_hbm, xs_hbm, idx_v, rowbuf, sem):
    wid = lax.axis_index("s") * NC + lax.axis_index("c")
    pltpu.sync_copy(dst_hbm.at[wid], idx_v)
    for c in range(NCH):
        base = wid * TPW + c * RCH
        pltpu.sync_copy(xsc_hbm.at[pl.ds(base, RCH)], rowbuf)
        pltpu.async_copy(rowbuf, xs_hbm.at[idx_v.at[c]], sem).wait()


_sc_scatter = functools.partial(
    pl.kernel,
    out_type=jax.ShapeDtypeStruct((SROWS, HIDDEN), jnp.float32),
    scratch_types=[
        pltpu.VMEM((NCH, RCH), jnp.int32),
        pltpu.VMEM((RCH, HIDDEN), jnp.float32),
        pltpu.SemaphoreType.DMA,
    ],
    mesh=plsc.VectorSubcoreMesh(core_axis_name="c", subcore_axis_name="s"),
)(_sc_scatter_body)


def _sc_gather_body(ys_hbm, dst_hbm, yr_hbm, idx_v, rowbuf, sem):
    wid = lax.axis_index("s") * NC + lax.axis_index("c")
    pltpu.sync_copy(dst_hbm.at[wid], idx_v)
    for c in range(NCH):
        base = wid * TPW + c * RCH
        pltpu.async_copy(ys_hbm.at[idx_v.at[c]], rowbuf, sem).wait()
        pltpu.sync_copy(rowbuf, yr_hbm.at[pl.ds(base, RCH)])


_sc_gather = functools.partial(
    pl.kernel,
    out_type=jax.ShapeDtypeStruct((T, HIDDEN), jnp.float32),
    scratch_types=[
        pltpu.VMEM((NCH, RCH), jnp.int32),
        pltpu.VMEM((RCH, HIDDEN), jnp.float32),
        pltpu.SemaphoreType.DMA,
    ],
    mesh=plsc.VectorSubcoreMesh(core_axis_name="c", subcore_axis_name="s"),
)(_sc_gather_body)


def _gemm_body(be_ref, nu_ref, xs_ref, w1g_ref, w1u_ref, w2_ref, ys_ref):
    b = pl.program_id(0)
    j = pl.program_id(1)

    @pl.when(b < nu_ref[0])
    def _():
        xb = xs_ref[...]
        g = jnp.dot(xb, w1g_ref[0], preferred_element_type=jnp.float32)
        u = jnp.dot(xb, w1u_ref[0], preferred_element_type=jnp.float32)
        h = g * jax.nn.sigmoid(g) * u
        p = jnp.dot(h, w2_ref[0], preferred_element_type=jnp.float32)

        @pl.when(j == 0)
        def _():
            ys_ref[...] = p

        @pl.when(j > 0)
        def _():
            ys_ref[...] += p


def _run_gemm(bexp, nu, xs, gate_up_proj, down_proj):
    return pl.pallas_call(
        _gemm_body,
        grid_spec=pltpu.PrefetchScalarGridSpec(
            num_scalar_prefetch=2,
            grid=(NBLK, NJ),
            in_specs=[
                pl.BlockSpec((BT, HIDDEN), lambda b, j, be, nu: (b, 0)),
                pl.BlockSpec((1, HIDDEN, CJ),
                             lambda b, j, be, nu: (be[b], 0, j)),
                pl.BlockSpec((1, HIDDEN, CJ),
                             lambda b, j, be, nu: (be[b], 0, NJ + j)),
                pl.BlockSpec((1, CJ, HIDDEN),
                             lambda b, j, be, nu: (be[b], j, 0)),
            ],
            out_specs=pl.BlockSpec((BT, HIDDEN), lambda b, j, be, nu: (b, 0)),
        ),
        out_shape=jax.ShapeDtypeStruct((SROWS, HIDDEN), jnp.float32),
        compiler_params=pltpu.CompilerParams(
            dimension_semantics=("arbitrary", "arbitrary")),
    )(bexp, nu, xs, gate_up_proj, gate_up_proj, down_proj)


def _shared_body(x_ref, gw_ref, uw_ref, dw_ref, yr_ref, out_ref):
    j = pl.program_id(1)
    xb = x_ref[...]
    g = jnp.dot(xb, gw_ref[...], preferred_element_type=jnp.float32)
    u = jnp.dot(xb, uw_ref[...], preferred_element_type=jnp.float32)
    h = g * jax.nn.sigmoid(g) * u
    p = jnp.dot(h, dw_ref[...], preferred_element_type=jnp.float32)

    @pl.when(j == 0)
    def _():
        out_ref[...] = p + yr_ref[...]

    @pl.when(j > 0)
    def _():
        out_ref[...] += p


def _run_shared(x, gw, uw, dw, yr):
    return pl.pallas_call(
        _shared_body,
        grid=(T // BT, NJ),
        in_specs=[
            pl.BlockSpec((BT, HIDDEN), lambda b, j: (b, 0)),
            pl.BlockSpec((HIDDEN, CJ), lambda b, j: (0, j)),
            pl.BlockSpec((HIDDEN, CJ), lambda b, j: (0, j)),
            pl.BlockSpec((CJ, HIDDEN), lambda b, j: (j, 0)),
            pl.BlockSpec((BT, HIDDEN), lambda b, j: (b, 0)),
        ],
        out_specs=pl.BlockSpec((BT, HIDDEN), lambda b, j: (b, 0)),
        out_shape=jax.ShapeDtypeStruct((T, HIDDEN), jnp.float32),
        compiler_params=pltpu.CompilerParams(
            dimension_semantics=("arbitrary", "arbitrary")),
    )(x, gw, uw, dw, yr)


def kernel(hidden_states, router_weight, gate_up_proj, down_proj,
           shared_gate_w, shared_up_w, shared_down_w):
    x = hidden_states.reshape(T, HIDDEN)
    wr_pad = jnp.pad(router_weight, ((0, 0), (0, 128 - E)))

    logits_pad, x_scaled, onehot = _run_router(x, wr_pad)
    dst, sched = _run_sched(onehot)
    dst3d = dst.reshape(NW, NCH, RCH)
    bexp = sched[:NBLK, 0]
    nu = sched[NBLK:NBLK + 1, 0]

    xs = _sc_scatter(x_scaled, dst3d)
    ys = _run_gemm(bexp, nu, xs, gate_up_proj, down_proj)
    yr = _sc_gather(ys, dst3d)
    out = _run_shared(x, shared_gate_w, shared_up_w, shared_down_w, yr)
    return out, logits_pad[:, :E]
